# baseline (speedup 1.0000x reference)
import functools

import jax
import jax.numpy as jnp
from jax import lax
from jax.experimental import pallas as pl
from jax.experimental.pallas import tpu as pltpu
from jax.experimental.pallas import tpu_sc as plsc

N = 5000
E = 160000
T = 10000
D = 128
NP = 5120
RPT = 320
GK = 80
GC_PER_TILE = 4
EK = 125
EC = E // EK
EC_PER_TILE = EC // 16

_MESH = plsc.VectorSubcoreMesh(core_axis_name="c", subcore_axis_name="s")


DK = 128
DC = 1280
DC_PER_TILE = DC // 16


def _sc_gather_deg(t_emb, t2e, t2n, dste, dstn,
                   g_e, g_n, deg_e, deg_n,
                   idx_v, rows_v, dix_v, ones_v, z320_v, deg_sh, sem):
    c = lax.axis_index("c")
    s = lax.axis_index("s")

    def fo(k, _):
        ones_v[pl.ds(k * 16, 16)] = jnp.ones((16,), jnp.float32)
        return 0
    lax.fori_loop(0, DK // 16, fo, 0)

    def fz(i, _):
        z320_v[pl.ds(i * 16, 16)] = jnp.zeros((16,), jnp.float32)
        return 0
    lax.fori_loop(0, RPT // 16, fz, 0)

    pltpu.sync_copy(z320_v, deg_sh.at[pl.ds(s * RPT, RPT)])

    def branch(t2x, dst_r, g_out, deg_out):
        pltpu.sync_copy(t2x.at[pl.ds(s * GC_PER_TILE, GC_PER_TILE)], idx_v)

        def gchunk(j, _):
            pltpu.async_copy(t_emb.at[idx_v.at[j]], rows_v, sem).wait()
            pltpu.sync_copy(rows_v, g_out.at[pl.ds((s * GC_PER_TILE + j) * GK, GK)])
            return 0
        lax.fori_loop(0, GC_PER_TILE, gchunk, 0)

        pltpu.sync_copy(dst_r.at[pl.ds(s * DC_PER_TILE, DC_PER_TILE)], dix_v)
        plsc.subcore_barrier()

        def dchunk(i, _):
            for b in range(5):
                pltpu.async_copy(ones_v, deg_sh.at[dix_v.at[i * 5 + b]], sem,
                                 add=True)
            for b in range(5):
                pltpu.make_async_copy(ones_v, deg_sh.at[dix_v.at[i * 5 + b]],
                                      sem).wait()
            return 0
        lax.fori_loop(0, DC_PER_TILE // 5, dchunk, 0)
        plsc.subcore_barrier()

        pltpu.sync_copy(deg_sh.at[pl.ds(s * RPT, RPT)], z320_v)
        pltpu.sync_copy(z320_v, deg_out.at[pl.ds(s * RPT, RPT)])

    @pl.when(c == 0)
    def _():
        branch(t2e, dste, g_e, deg_e)

    @pl.when(c == 1)
    def _():
        branch(t2n, dstn, g_n, deg_n)


_sc_phase1 = functools.partial(
    pl.kernel,
    mesh=_MESH,
    out_type=[
        jax.ShapeDtypeStruct((NP, D), jnp.float32),
        jax.ShapeDtypeStruct((NP, D), jnp.float32),
        jax.ShapeDtypeStruct((NP,), jnp.float32),
        jax.ShapeDtypeStruct((NP,), jnp.float32),
    ],
    scratch_types=[
        pltpu.VMEM((GC_PER_TILE, GK), jnp.int32),
        pltpu.VMEM((GK, D), jnp.float32),
        pltpu.VMEM((DC_PER_TILE, DK), jnp.int32),
        pltpu.VMEM((DK,), jnp.float32),
        pltpu.VMEM((RPT,), jnp.float32),
        pltpu.VMEM_SHARED((NP,), jnp.float32),
        pltpu.SemaphoreType.DMA,
    ],
)(_sc_gather_deg)


def _sc_scatter(y_e, y_n, srce, dste, srcn, dstn,
                acc_e, acc_n,
                six_v, dix_v, rows_v, zrow_v, acc_sh, sem, ssem0, ssem1):
    c = lax.axis_index("c")
    s = lax.axis_index("s")

    def fz(i, _):
        for k in range(D // 16):
            zrow_v[i, pl.ds(k * 16, 16)] = jnp.zeros((16,), jnp.float32)
        return 0
    lax.fori_loop(0, 64, fz, 0)
    for q in range(RPT // 64):
        pltpu.sync_copy(zrow_v, acc_sh.at[pl.ds(s * RPT + q * 64, 64)])

    def branch(y, src_r, dst_r, acc_out):
        pltpu.sync_copy(src_r.at[pl.ds(s * EC_PER_TILE, EC_PER_TILE)], six_v)
        pltpu.sync_copy(dst_r.at[pl.ds(s * EC_PER_TILE, EC_PER_TILE)], dix_v)
        plsc.subcore_barrier()

        pltpu.async_copy(y.at[six_v.at[0]], rows_v.at[0], sem)

        def echunk(i, _):
            for b in range(2):
                j = i * 2 + b
                ssem = ssem0 if b == 0 else ssem1
                osem = ssem1 if b == 0 else ssem0
                pltpu.make_async_copy(y.at[six_v.at[j]], rows_v.at[b],
                                      sem).wait()
                pltpu.async_copy(rows_v.at[b], acc_sh.at[dix_v.at[j]], ssem,
                                 add=True)

                @pl.when(j >= 1)
                def _():
                    pltpu.make_async_copy(rows_v.at[1 - b],
                                          acc_sh.at[dix_v.at[j - 1]],
                                          osem).wait()

                @pl.when(j + 1 < EC_PER_TILE)
                def _():
                    pltpu.async_copy(y.at[six_v.at[j + 1]], rows_v.at[1 - b],
                                     sem)
            return 0
        lax.fori_loop(0, EC_PER_TILE // 2, echunk, 0)
        pltpu.make_async_copy(rows_v.at[1],
                              acc_sh.at[dix_v.at[EC_PER_TILE - 1]],
                              ssem1).wait()
        plsc.subcore_barrier()

        for q in range(RPT // 64):
            pltpu.sync_copy(acc_sh.at[pl.ds(s * RPT + q * 64, 64)], zrow_v)
            pltpu.sync_copy(zrow_v, acc_out.at[pl.ds(s * RPT + q * 64, 64)])

    @pl.when(c == 0)
    def _():
        branch(y_e, srce, dste, acc_e)

    @pl.when(c == 1)
    def _():
        branch(y_n, srcn, dstn, acc_n)


_sc_phase2 = functools.partial(
    pl.kernel,
    mesh=_MESH,
    out_type=[
        jax.ShapeDtypeStruct((NP, D), jnp.float32),
        jax.ShapeDtypeStruct((NP, D), jnp.float32),
    ],
    scratch_types=[
        pltpu.VMEM((EC_PER_TILE, EK), jnp.int32),
        pltpu.VMEM((EC_PER_TILE, EK), jnp.int32),
        pltpu.VMEM((2, EK, D), jnp.float32),
        pltpu.VMEM((64, D), jnp.float32),
        pltpu.VMEM_SHARED((NP, D), jnp.float32),
        pltpu.SemaphoreType.DMA,
        pltpu.SemaphoreType.DMA,
        pltpu.SemaphoreType.DMA,
    ],
)(_sc_scatter)


def _mm_body(g_ref, w_ref, deg_ref, y_ref):
    xw = jnp.dot(g_ref[...], w_ref[...], preferred_element_type=jnp.float32)
    dis = lax.rsqrt(deg_ref[...] + 1.0)
    y_ref[...] = xw * dis


def _mm(g, W, deg):
    return pl.pallas_call(
        _mm_body,
        grid=(8,),
        in_specs=[
            pl.BlockSpec((NP // 8, D), lambda i: (i, 0)),
            pl.BlockSpec((D, D), lambda i: (0, 0)),
            pl.BlockSpec((NP // 8, 1), lambda i: (i, 0)),
        ],
        out_specs=pl.BlockSpec((NP // 8, D), lambda i: (i, 0)),
        out_shape=jax.ShapeDtypeStruct((NP, D), jnp.float32),
    )(g, W, deg)


_CB = 1000


def _comb_body(te_ref, ye_ref, yn_ref, ae_ref, an_ref, de_ref, dn_ref,
               b2_ref, g2_ref, out_ref):
    i = pl.program_id(0)
    ta = jnp.tanh(g2_ref[0, 0])
    tb = jnp.tanh(g2_ref[0, 1])
    dis_e = lax.rsqrt(de_ref[...] + 1.0)
    dis_n = lax.rsqrt(dn_ref[...] + 1.0)
    ce = dis_e * (ae_ref[...] + ye_ref[...]) + b2_ref[0:1, :]
    cn = dis_n * (an_ref[...] + yn_ref[...]) + b2_ref[1:2, :]
    row = i * _CB + lax.broadcasted_iota(jnp.int32, (_CB, 1), 0)
    out_ref[...] = te_ref[...] + jnp.where(row < N, ta * ce + tb * cn, 0.0)


def _combine(te, ye, yn, ae, an, de, dn, b2, g2):
    nb = N // _CB
    cap = lambda i: (jnp.minimum(i, nb - 1), 0)
    return pl.pallas_call(
        _comb_body,
        grid=(T // _CB,),
        in_specs=[
            pl.BlockSpec((_CB, D), lambda i: (i, 0)),
            pl.BlockSpec((_CB, D), cap),
            pl.BlockSpec((_CB, D), cap),
            pl.BlockSpec((_CB, D), cap),
            pl.BlockSpec((_CB, D), cap),
            pl.BlockSpec((_CB, 1), cap),
            pl.BlockSpec((_CB, 1), cap),
            pl.BlockSpec((2, D), lambda i: (0, 0)),
            pl.BlockSpec((1, 2), lambda i: (0, 0)),
        ],
        out_specs=pl.BlockSpec((_CB, D), lambda i: (i, 0)),
        out_shape=jax.ShapeDtypeStruct((T, D), jnp.float32),
    )(te, ye, yn, ae, an, de, dn, b2, g2)


def kernel(token_embeddings, tokens2edges, edge_index_edges, edges2tokens,
           tokens2nodes, edge_index_nodes, nodes2tokens,
           W_edges, b_edges, W_nodes, b_nodes, gate_a, gate_b):
    te = token_embeddings[0]
    pad = jnp.zeros((NP - N,), jnp.int32)
    t2e = jnp.concatenate([tokens2edges, pad]).reshape(NP // GK, GK)
    t2n = jnp.concatenate([tokens2nodes, pad]).reshape(NP // GK, GK)
    srce = edge_index_edges[0].reshape(EC, EK)
    dste = edge_index_edges[1].reshape(EC, EK)
    srcn = edge_index_nodes[0].reshape(EC, EK)
    dstn = edge_index_nodes[1].reshape(EC, EK)

    dpad = jnp.full((DC * DK - E,), NP - 1, jnp.int32)
    dste_d = jnp.concatenate([edge_index_edges[1], dpad]).reshape(DC, DK)
    dstn_d = jnp.concatenate([edge_index_nodes[1], dpad]).reshape(DC, DK)
    g_e, g_n, deg_e, deg_n = _sc_phase1(te, t2e, t2n, dste_d, dstn_d)
    deg_e = deg_e[:, None]
    deg_n = deg_n[:, None]
    y_e = _mm(g_e, W_edges, deg_e)
    y_n = _mm(g_n, W_nodes, deg_n)
    acc_e, acc_n = _sc_phase2(y_e, y_n, srce, dste, srcn, dstn)

    b2 = jnp.stack([b_edges, b_nodes])
    g2 = jnp.concatenate([gate_a, gate_b])[None, :]
    out = _combine(te, y_e[:N], y_n[:N], acc_e[:N], acc_n[:N],
                   deg_e[:N], deg_n[:N], b2, g2)
    return out[None]


# speedup vs baseline: 1.0980x; 1.0980x over previous
import functools

import jax
import jax.numpy as jnp
from jax import lax
from jax.experimental import pallas as pl
from jax.experimental.pallas import tpu as pltpu
from jax.experimental.pallas import tpu_sc as plsc

N = 5000
E = 160000
T = 10000
D = 128
NP = 5120
RPT = 320
GK = 80
GC_PER_TILE = 4
EK = 125
EC = E // EK
EC_PER_TILE = EC // 16

_MESH = plsc.VectorSubcoreMesh(core_axis_name="c", subcore_axis_name="s")


DK = 128
DC = 1280
DC_PER_TILE = DC // 16


def _sc_gather_deg(t_emb, t2e, t2n, dste, dstn,
                   g_e, g_n, deg_e, deg_n,
                   idx_v, rows_v, dix_v, ones_v, z320_v, deg_sh, sem):
    c = lax.axis_index("c")
    s = lax.axis_index("s")

    def fo(k, _):
        ones_v[pl.ds(k * 16, 16)] = jnp.ones((16,), jnp.float32)
        return 0
    lax.fori_loop(0, DK // 16, fo, 0)

    def fz(i, _):
        z320_v[pl.ds(i * 16, 16)] = jnp.zeros((16,), jnp.float32)
        return 0
    lax.fori_loop(0, RPT // 16, fz, 0)

    pltpu.sync_copy(z320_v, deg_sh.at[pl.ds(s * RPT, RPT)])

    def branch(t2x, dst_r, g_out, deg_out):
        pltpu.sync_copy(t2x.at[pl.ds(s * GC_PER_TILE, GC_PER_TILE)], idx_v)

        def gchunk(j, _):
            pltpu.async_copy(t_emb.at[idx_v.at[j]], rows_v, sem).wait()
            pltpu.sync_copy(rows_v, g_out.at[pl.ds((s * GC_PER_TILE + j) * GK, GK)])
            return 0
        lax.fori_loop(0, GC_PER_TILE, gchunk, 0)

        pltpu.sync_copy(dst_r.at[pl.ds(s * DC_PER_TILE, DC_PER_TILE)], dix_v)
        plsc.subcore_barrier()

        def dchunk(i, _):
            for b in range(5):
                pltpu.async_copy(ones_v, deg_sh.at[dix_v.at[i * 5 + b]], sem,
                                 add=True)
            for b in range(5):
                pltpu.make_async_copy(ones_v, deg_sh.at[dix_v.at[i * 5 + b]],
                                      sem).wait()
            return 0
        lax.fori_loop(0, DC_PER_TILE // 5, dchunk, 0)
        plsc.subcore_barrier()

        pltpu.sync_copy(deg_sh.at[pl.ds(s * RPT, RPT)], z320_v)
        pltpu.sync_copy(z320_v, deg_out.at[pl.ds(s * RPT, RPT)])

    @pl.when(c == 0)
    def _():
        branch(t2e, dste, g_e, deg_e)

    @pl.when(c == 1)
    def _():
        branch(t2n, dstn, g_n, deg_n)


_sc_phase1 = functools.partial(
    pl.kernel,
    mesh=_MESH,
    out_type=[
        jax.ShapeDtypeStruct((NP, D), jnp.float32),
        jax.ShapeDtypeStruct((NP, D), jnp.float32),
        jax.ShapeDtypeStruct((NP,), jnp.float32),
        jax.ShapeDtypeStruct((NP,), jnp.float32),
    ],
    scratch_types=[
        pltpu.VMEM((GC_PER_TILE, GK), jnp.int32),
        pltpu.VMEM((GK, D), jnp.float32),
        pltpu.VMEM((DC_PER_TILE, DK), jnp.int32),
        pltpu.VMEM((DK,), jnp.float32),
        pltpu.VMEM((RPT,), jnp.float32),
        pltpu.VMEM_SHARED((NP,), jnp.float32),
        pltpu.SemaphoreType.DMA,
    ],
)(_sc_gather_deg)


def _sc_scatter(y_e, y_n, srce, dste, srcn, dstn,
                acc_e, acc_n,
                six_v, dix_v, rows_v, acc_sh,
                gsem0, gsem1, ssem0, ssem1, ssem2, ssem3):
    c = lax.axis_index("c")
    s = lax.axis_index("s")
    gsems = [gsem0, gsem1]
    ssems = [ssem0, ssem1, ssem2, ssem3]

    def fz(i, _):
        for k in range(D // 16):
            rows_v[0, i, pl.ds(k * 16, 16)] = jnp.zeros((16,), jnp.float32)
        return 0
    lax.fori_loop(0, 64, fz, 0)
    for q in range(RPT // 64):
        pltpu.sync_copy(rows_v.at[0, pl.ds(0, 64)],
                        acc_sh.at[pl.ds(s * RPT + q * 64, 64)])

    def branch(y, src_r, dst_r, acc_out):
        pltpu.sync_copy(src_r.at[pl.ds(s * EC_PER_TILE, EC_PER_TILE)], six_v)
        pltpu.sync_copy(dst_r.at[pl.ds(s * EC_PER_TILE, EC_PER_TILE)], dix_v)
        plsc.subcore_barrier()

        pltpu.async_copy(y.at[six_v.at[0]], rows_v.at[0], gsems[0])
        pltpu.async_copy(y.at[six_v.at[1]], rows_v.at[1], gsems[1])

        def echunk(i, _):
            for b in range(4):
                j = i * 4 + b
                pltpu.make_async_copy(y.at[six_v.at[j]], rows_v.at[b],
                                      gsems[b % 2]).wait()
                pltpu.async_copy(rows_v.at[b], acc_sh.at[dix_v.at[j]],
                                 ssems[b], add=True)
                nb = (b + 2) % 4

                @pl.when(j >= 2)
                def _():
                    pltpu.make_async_copy(rows_v.at[nb],
                                          acc_sh.at[dix_v.at[j - 2]],
                                          ssems[nb]).wait()

                @pl.when(j + 2 < EC_PER_TILE)
                def _():
                    pltpu.async_copy(y.at[six_v.at[j + 2]], rows_v.at[nb],
                                     gsems[b % 2])
            return 0
        lax.fori_loop(0, EC_PER_TILE // 4, echunk, 0)
        pltpu.make_async_copy(rows_v.at[2],
                              acc_sh.at[dix_v.at[EC_PER_TILE - 2]],
                              ssems[2]).wait()
        pltpu.make_async_copy(rows_v.at[3],
                              acc_sh.at[dix_v.at[EC_PER_TILE - 1]],
                              ssems[3]).wait()
        plsc.subcore_barrier()

        pltpu.sync_copy(acc_sh.at[pl.ds(s * RPT, RPT)],
                        acc_out.at[pl.ds(s * RPT, RPT)])

    @pl.when(c == 0)
    def _():
        branch(y_e, srce, dste, acc_e)

    @pl.when(c == 1)
    def _():
        branch(y_n, srcn, dstn, acc_n)


_sc_phase2 = functools.partial(
    pl.kernel,
    mesh=_MESH,
    out_type=[
        jax.ShapeDtypeStruct((NP, D), jnp.float32),
        jax.ShapeDtypeStruct((NP, D), jnp.float32),
    ],
    scratch_types=[
        pltpu.VMEM((EC_PER_TILE, EK), jnp.int32),
        pltpu.VMEM((EC_PER_TILE, EK), jnp.int32),
        pltpu.VMEM((4, EK, D), jnp.float32),
        pltpu.VMEM_SHARED((NP, D), jnp.float32),
        pltpu.SemaphoreType.DMA,
        pltpu.SemaphoreType.DMA,
        pltpu.SemaphoreType.DMA,
        pltpu.SemaphoreType.DMA,
        pltpu.SemaphoreType.DMA,
        pltpu.SemaphoreType.DMA,
    ],
)(_sc_scatter)


def _mm_body(g_ref, w_ref, deg_ref, y_ref):
    xw = jnp.dot(g_ref[...], w_ref[...], preferred_element_type=jnp.float32)
    dis = lax.rsqrt(deg_ref[...] + 1.0)
    y_ref[...] = xw * dis


def _mm(g, W, deg):
    return pl.pallas_call(
        _mm_body,
        grid=(8,),
        in_specs=[
            pl.BlockSpec((NP // 8, D), lambda i: (i, 0)),
            pl.BlockSpec((D, D), lambda i: (0, 0)),
            pl.BlockSpec((NP // 8, 1), lambda i: (i, 0)),
        ],
        out_specs=pl.BlockSpec((NP // 8, D), lambda i: (i, 0)),
        out_shape=jax.ShapeDtypeStruct((NP, D), jnp.float32),
    )(g, W, deg)


_CB = 1000


def _comb_body(te_ref, ye_ref, yn_ref, ae_ref, an_ref, de_ref, dn_ref,
               b2_ref, g2_ref, out_ref):
    i = pl.program_id(0)
    ta = jnp.tanh(g2_ref[0, 0])
    tb = jnp.tanh(g2_ref[0, 1])
    dis_e = lax.rsqrt(de_ref[...] + 1.0)
    dis_n = lax.rsqrt(dn_ref[...] + 1.0)
    ce = dis_e * (ae_ref[...] + ye_ref[...]) + b2_ref[0:1, :]
    cn = dis_n * (an_ref[...] + yn_ref[...]) + b2_ref[1:2, :]
    row = i * _CB + lax.broadcasted_iota(jnp.int32, (_CB, 1), 0)
    out_ref[...] = te_ref[...] + jnp.where(row < N, ta * ce + tb * cn, 0.0)


def _combine(te, ye, yn, ae, an, de, dn, b2, g2):
    nb = N // _CB
    cap = lambda i: (jnp.minimum(i, nb - 1), 0)
    return pl.pallas_call(
        _comb_body,
        grid=(T // _CB,),
        in_specs=[
            pl.BlockSpec((_CB, D), lambda i: (i, 0)),
            pl.BlockSpec((_CB, D), cap),
            pl.BlockSpec((_CB, D), cap),
            pl.BlockSpec((_CB, D), cap),
            pl.BlockSpec((_CB, D), cap),
            pl.BlockSpec((_CB, 1), cap),
            pl.BlockSpec((_CB, 1), cap),
            pl.BlockSpec((2, D), lambda i: (0, 0)),
            pl.BlockSpec((1, 2), lambda i: (0, 0)),
        ],
        out_specs=pl.BlockSpec((_CB, D), lambda i: (i, 0)),
        out_shape=jax.ShapeDtypeStruct((T, D), jnp.float32),
    )(te, ye, yn, ae, an, de, dn, b2, g2)


def kernel(token_embeddings, tokens2edges, edge_index_edges, edges2tokens,
           tokens2nodes, edge_index_nodes, nodes2tokens,
           W_edges, b_edges, W_nodes, b_nodes, gate_a, gate_b):
    te = token_embeddings[0]
    pad = jnp.zeros((NP - N,), jnp.int32)
    t2e = jnp.concatenate([tokens2edges, pad]).reshape(NP // GK, GK)
    t2n = jnp.concatenate([tokens2nodes, pad]).reshape(NP // GK, GK)
    srce = edge_index_edges[0].reshape(EC, EK)
    dste = edge_index_edges[1].reshape(EC, EK)
    srcn = edge_index_nodes[0].reshape(EC, EK)
    dstn = edge_index_nodes[1].reshape(EC, EK)

    dpad = jnp.full((DC * DK - E,), NP - 1, jnp.int32)
    dste_d = jnp.concatenate([edge_index_edges[1], dpad]).reshape(DC, DK)
    dstn_d = jnp.concatenate([edge_index_nodes[1], dpad]).reshape(DC, DK)
    g_e, g_n, deg_e, deg_n = _sc_phase1(te, t2e, t2n, dste_d, dstn_d)
    deg_e = deg_e[:, None]
    deg_n = deg_n[:, None]
    y_e = _mm(g_e, W_edges, deg_e)
    y_n = _mm(g_n, W_nodes, deg_n)
    acc_e, acc_n = _sc_phase2(y_e, y_n, srce, dste, srcn, dstn)

    b2 = jnp.stack([b_edges, b_nodes])
    g2 = jnp.concatenate([gate_a, gate_b])[None, :]
    out = _combine(te, y_e[:N], y_n[:N], acc_e[:N], acc_n[:N],
                   deg_e[:N], deg_n[:N], b2, g2)
    return out[None]


# speedup vs baseline: 1.1353x; 1.0339x over previous
import functools

import jax
import jax.numpy as jnp
from jax import lax
from jax.experimental import pallas as pl
from jax.experimental.pallas import tpu as pltpu
from jax.experimental.pallas import tpu_sc as plsc

N = 5000
E = 160000
T = 10000
D = 128
NP = 5120
RPT = 320
GK = 80
GC_PER_TILE = 4
EK = 125
EC = E // EK
EC_PER_TILE = EC // 16

_MESH = plsc.VectorSubcoreMesh(core_axis_name="c", subcore_axis_name="s")


DK = 128
DC = 1280
DC_PER_TILE = DC // 16


def _sc_gather_deg(t_emb, t2e, t2n, dste, dstn,
                   g2, deg_e, deg_n,
                   idx_v, rows_v, dix_v, ones_v, z320_v, deg_sh, sem, sem2):
    c = lax.axis_index("c")
    s = lax.axis_index("s")

    def fo(k, _):
        ones_v[pl.ds(k * 16, 16)] = jnp.ones((16,), jnp.float32)
        return 0
    lax.fori_loop(0, DK // 16, fo, 0)

    def fz(i, _):
        z320_v[pl.ds(i * 16, 16)] = jnp.zeros((16,), jnp.float32)
        return 0
    lax.fori_loop(0, RPT // 16, fz, 0)

    pltpu.sync_copy(z320_v, deg_sh.at[pl.ds(s * RPT, RPT)])

    def branch(t2x, dst_r, g_out, deg_out):
        pltpu.sync_copy(t2x.at[pl.ds(s * GC_PER_TILE, GC_PER_TILE)], idx_v)
        pltpu.async_copy(t_emb.at[idx_v.at[0]], rows_v.at[0], sem)
        for j in range(GC_PER_TILE):
            pltpu.make_async_copy(t_emb.at[idx_v.at[j]], rows_v.at[j % 2],
                                  sem).wait()
            if j >= 1:
                pltpu.make_async_copy(
                    rows_v.at[(j - 1) % 2],
                    g_out.at[pl.ds((s * GC_PER_TILE + j - 1) * GK, GK)],
                    sem2).wait()
            if j + 1 < GC_PER_TILE:
                pltpu.async_copy(t_emb.at[idx_v.at[j + 1]],
                                 rows_v.at[(j + 1) % 2], sem)
            pltpu.async_copy(rows_v.at[j % 2],
                             g_out.at[pl.ds((s * GC_PER_TILE + j) * GK, GK)],
                             sem2)
        jlast = GC_PER_TILE - 1
        pltpu.make_async_copy(
            rows_v.at[jlast % 2],
            g_out.at[pl.ds((s * GC_PER_TILE + jlast) * GK, GK)], sem2).wait()

        pltpu.sync_copy(dst_r.at[pl.ds(s * DC_PER_TILE, DC_PER_TILE)], dix_v)
        plsc.subcore_barrier()

        def dchunk(i, _):
            for b in range(5):
                pltpu.async_copy(ones_v, deg_sh.at[dix_v.at[i * 5 + b]], sem,
                                 add=True)
            for b in range(5):
                pltpu.make_async_copy(ones_v, deg_sh.at[dix_v.at[i * 5 + b]],
                                      sem).wait()
            return 0
        lax.fori_loop(0, DC_PER_TILE // 5, dchunk, 0)
        plsc.subcore_barrier()

        pltpu.sync_copy(deg_sh.at[pl.ds(s * RPT, RPT)], z320_v)
        pltpu.sync_copy(z320_v, deg_out.at[pl.ds(s * RPT, RPT)])

    @pl.when(c == 0)
    def _():
        branch(t2e, dste, g2.at[0], deg_e)

    @pl.when(c == 1)
    def _():
        branch(t2n, dstn, g2.at[1], deg_n)


_sc_phase1 = functools.partial(
    pl.kernel,
    mesh=_MESH,
    out_type=[
        jax.ShapeDtypeStruct((2, NP, D), jnp.float32),
        jax.ShapeDtypeStruct((NP,), jnp.float32),
        jax.ShapeDtypeStruct((NP,), jnp.float32),
    ],
    scratch_types=[
        pltpu.VMEM((GC_PER_TILE, GK), jnp.int32),
        pltpu.VMEM((2, GK, D), jnp.float32),
        pltpu.VMEM((DC_PER_TILE, DK), jnp.int32),
        pltpu.VMEM((DK,), jnp.float32),
        pltpu.VMEM((RPT,), jnp.float32),
        pltpu.VMEM_SHARED((NP,), jnp.float32),
        pltpu.SemaphoreType.DMA,
        pltpu.SemaphoreType.DMA,
    ],
)(_sc_gather_deg)


def _sc_scatter(y2, srce, dste, srcn, dstn,
                acc2,
                six_v, dix_v, rows_v, acc_sh,
                gsem0, gsem1, ssem0, ssem1, ssem2, ssem3):
    c = lax.axis_index("c")
    s = lax.axis_index("s")
    gsems = [gsem0, gsem1]
    ssems = [ssem0, ssem1, ssem2, ssem3]

    def fz(i, _):
        for k in range(D // 16):
            rows_v[0, i, pl.ds(k * 16, 16)] = jnp.zeros((16,), jnp.float32)
        return 0
    lax.fori_loop(0, 64, fz, 0)
    for q in range(RPT // 64):
        pltpu.sync_copy(rows_v.at[0, pl.ds(0, 64)],
                        acc_sh.at[pl.ds(s * RPT + q * 64, 64)])

    def branch(y, src_r, dst_r, acc_out):
        pltpu.sync_copy(src_r.at[pl.ds(s * EC_PER_TILE, EC_PER_TILE)], six_v)
        pltpu.sync_copy(dst_r.at[pl.ds(s * EC_PER_TILE, EC_PER_TILE)], dix_v)
        plsc.subcore_barrier()

        pltpu.async_copy(y.at[six_v.at[0]], rows_v.at[0], gsems[0])
        pltpu.async_copy(y.at[six_v.at[1]], rows_v.at[1], gsems[1])

        def echunk(i, _):
            for b in range(4):
                j = i * 4 + b
                pltpu.make_async_copy(y.at[six_v.at[j]], rows_v.at[b],
                                      gsems[b % 2]).wait()
                pltpu.async_copy(rows_v.at[b], acc_sh.at[dix_v.at[j]],
                                 ssems[b], add=True)
                nb = (b + 2) % 4

                @pl.when(j >= 2)
                def _():
                    pltpu.make_async_copy(rows_v.at[nb],
                                          acc_sh.at[dix_v.at[j - 2]],
                                          ssems[nb]).wait()

                @pl.when(j + 2 < EC_PER_TILE)
                def _():
                    pltpu.async_copy(y.at[six_v.at[j + 2]], rows_v.at[nb],
                                     gsems[b % 2])
            return 0
        lax.fori_loop(0, EC_PER_TILE // 4, echunk, 0)
        pltpu.make_async_copy(rows_v.at[2],
                              acc_sh.at[dix_v.at[EC_PER_TILE - 2]],
                              ssems[2]).wait()
        pltpu.make_async_copy(rows_v.at[3],
                              acc_sh.at[dix_v.at[EC_PER_TILE - 1]],
                              ssems[3]).wait()
        plsc.subcore_barrier()

        pltpu.sync_copy(acc_sh.at[pl.ds(s * RPT, RPT)],
                        acc_out.at[pl.ds(s * RPT, RPT)])

    @pl.when(c == 0)
    def _():
        branch(y2.at[0], srce, dste, acc2.at[0])

    @pl.when(c == 1)
    def _():
        branch(y2.at[1], srcn, dstn, acc2.at[1])


_sc_phase2 = functools.partial(
    pl.kernel,
    mesh=_MESH,
    out_type=[
        jax.ShapeDtypeStruct((2, NP, D), jnp.float32),
    ],
    scratch_types=[
        pltpu.VMEM((EC_PER_TILE, EK), jnp.int32),
        pltpu.VMEM((EC_PER_TILE, EK), jnp.int32),
        pltpu.VMEM((4, EK, D), jnp.float32),
        pltpu.VMEM_SHARED((NP, D), jnp.float32),
        pltpu.SemaphoreType.DMA,
        pltpu.SemaphoreType.DMA,
        pltpu.SemaphoreType.DMA,
        pltpu.SemaphoreType.DMA,
        pltpu.SemaphoreType.DMA,
        pltpu.SemaphoreType.DMA,
    ],
)(_sc_scatter)


def _mm_body(g_ref, w_ref, deg_ref, y_ref):
    xw = jnp.dot(g_ref[0], w_ref[0], preferred_element_type=jnp.float32)
    dis = lax.rsqrt(deg_ref[0] + 1.0)
    y_ref[0] = xw * dis


def _mm(g2, W2, deg2):
    return pl.pallas_call(
        _mm_body,
        grid=(2, 8),
        in_specs=[
            pl.BlockSpec((1, NP // 8, D), lambda c, i: (c, i, 0)),
            pl.BlockSpec((1, D, D), lambda c, i: (c, 0, 0)),
            pl.BlockSpec((1, NP // 8, 1), lambda c, i: (c, i, 0)),
        ],
        out_specs=pl.BlockSpec((1, NP // 8, D), lambda c, i: (c, i, 0)),
        out_shape=jax.ShapeDtypeStruct((2, NP, D), jnp.float32),
    )(g2, W2, deg2)


_CB = 1000


def _comb_body(te_ref, ye_ref, yn_ref, ae_ref, an_ref, de_ref, dn_ref,
               b2_ref, gg_ref, out_ref):
    i = pl.program_id(0)
    ta = jnp.tanh(gg_ref[0, 0])
    tb = jnp.tanh(gg_ref[0, 1])
    dis_e = lax.rsqrt(de_ref[0] + 1.0)
    dis_n = lax.rsqrt(dn_ref[0] + 1.0)
    ce = dis_e * (ae_ref[0] + ye_ref[0]) + b2_ref[0:1, :]
    cn = dis_n * (an_ref[0] + yn_ref[0]) + b2_ref[1:2, :]
    row = i * _CB + lax.broadcasted_iota(jnp.int32, (_CB, 1), 0)
    out_ref[...] = te_ref[...] + jnp.where(row < N, ta * ce + tb * cn, 0.0)


def _combine(te, y2, acc2, deg2, b2, gg):
    nb = N // _CB
    cap_e = lambda i: (0, jnp.minimum(i, nb - 1), 0)
    cap_n = lambda i: (1, jnp.minimum(i, nb - 1), 0)
    return pl.pallas_call(
        _comb_body,
        grid=(T // _CB,),
        in_specs=[
            pl.BlockSpec((_CB, D), lambda i: (i, 0)),
            pl.BlockSpec((1, _CB, D), cap_e),
            pl.BlockSpec((1, _CB, D), cap_n),
            pl.BlockSpec((1, _CB, D), cap_e),
            pl.BlockSpec((1, _CB, D), cap_n),
            pl.BlockSpec((1, _CB, 1), cap_e),
            pl.BlockSpec((1, _CB, 1), cap_n),
            pl.BlockSpec((2, D), lambda i: (0, 0)),
            pl.BlockSpec((1, 2), lambda i: (0, 0)),
        ],
        out_specs=pl.BlockSpec((_CB, D), lambda i: (i, 0)),
        out_shape=jax.ShapeDtypeStruct((T, D), jnp.float32),
    )(te, y2, y2, acc2, acc2, deg2, deg2, b2, gg)


def kernel(token_embeddings, tokens2edges, edge_index_edges, edges2tokens,
           tokens2nodes, edge_index_nodes, nodes2tokens,
           W_edges, b_edges, W_nodes, b_nodes, gate_a, gate_b):
    te = token_embeddings[0]
    pad = jnp.zeros((NP - N,), jnp.int32)
    t2e = jnp.concatenate([tokens2edges, pad]).reshape(NP // GK, GK)
    t2n = jnp.concatenate([tokens2nodes, pad]).reshape(NP // GK, GK)
    srce = edge_index_edges[0].reshape(EC, EK)
    dste = edge_index_edges[1].reshape(EC, EK)
    srcn = edge_index_nodes[0].reshape(EC, EK)
    dstn = edge_index_nodes[1].reshape(EC, EK)

    dpad = jnp.full((DC * DK - E,), NP - 1, jnp.int32)
    dste_d = jnp.concatenate([edge_index_edges[1], dpad]).reshape(DC, DK)
    dstn_d = jnp.concatenate([edge_index_nodes[1], dpad]).reshape(DC, DK)
    g2, deg_e, deg_n = _sc_phase1(te, t2e, t2n, dste_d, dstn_d)
    deg2 = jnp.stack([deg_e, deg_n])[:, :, None]
    W2 = jnp.stack([W_edges, W_nodes])
    y2 = _mm(g2, W2, deg2)
    (acc2,) = _sc_phase2(y2, srce, dste, srcn, dstn)

    b2 = jnp.stack([b_edges, b_nodes])
    gg = jnp.concatenate([gate_a, gate_b])[None, :]
    out = _combine(te, y2, acc2, deg2, b2, gg)
    return out[None]


# speedup vs baseline: 1.1946x; 1.0522x over previous
import functools

import jax
import jax.numpy as jnp
from jax import lax
from jax.experimental import pallas as pl
from jax.experimental.pallas import tpu as pltpu
from jax.experimental.pallas import tpu_sc as plsc

N = 5000
E = 160000
T = 10000
D = 128
NP = 5120
RPT = 320
GK = 80
GC_PER_TILE = 4
EK = 125
EC = E // EK
EC_PER_TILE = EC // 16

_MESH = plsc.VectorSubcoreMesh(core_axis_name="c", subcore_axis_name="s")


DK = 128
DC = 1280
DC_PER_TILE = DC // 16


def _sc_gather_deg(t_emb, t2e, t2n, dste, dstn,
                   g2, deg_e, deg_n,
                   idx_v, rows_v, dix_v, ones_v, z320_v, deg_sh, sem, sem2):
    c = lax.axis_index("c")
    s = lax.axis_index("s")

    def fo(k, _):
        ones_v[pl.ds(k * 16, 16)] = jnp.ones((16,), jnp.float32)
        return 0
    lax.fori_loop(0, DK // 16, fo, 0)

    def fz(i, _):
        z320_v[pl.ds(i * 16, 16)] = jnp.zeros((16,), jnp.float32)
        return 0
    lax.fori_loop(0, RPT // 16, fz, 0)

    pltpu.sync_copy(z320_v, deg_sh.at[pl.ds(s * RPT, RPT)])

    def branch(t2x, dst_r, g_out, deg_out):
        pltpu.sync_copy(t2x.at[pl.ds(s * GC_PER_TILE, GC_PER_TILE)], idx_v)
        pltpu.async_copy(t_emb.at[idx_v.at[0]], rows_v.at[0], sem)
        for j in range(GC_PER_TILE):
            pltpu.make_async_copy(t_emb.at[idx_v.at[j]], rows_v.at[j % 2],
                                  sem).wait()
            if j >= 1:
                pltpu.make_async_copy(
                    rows_v.at[(j - 1) % 2],
                    g_out.at[pl.ds((s * GC_PER_TILE + j - 1) * GK, GK)],
                    sem2).wait()
            if j + 1 < GC_PER_TILE:
                pltpu.async_copy(t_emb.at[idx_v.at[j + 1]],
                                 rows_v.at[(j + 1) % 2], sem)
            pltpu.async_copy(rows_v.at[j % 2],
                             g_out.at[pl.ds((s * GC_PER_TILE + j) * GK, GK)],
                             sem2)
        jlast = GC_PER_TILE - 1
        pltpu.make_async_copy(
            rows_v.at[jlast % 2],
            g_out.at[pl.ds((s * GC_PER_TILE + jlast) * GK, GK)], sem2).wait()

        pltpu.sync_copy(dst_r.at[pl.ds(s * DC_PER_TILE, DC_PER_TILE)], dix_v)
        plsc.subcore_barrier()

        def dchunk(i, _):
            for b in range(5):
                pltpu.async_copy(ones_v, deg_sh.at[dix_v.at[i * 5 + b]], sem,
                                 add=True)
            for b in range(5):
                pltpu.make_async_copy(ones_v, deg_sh.at[dix_v.at[i * 5 + b]],
                                      sem).wait()
            return 0
        lax.fori_loop(0, DC_PER_TILE // 5, dchunk, 0)
        plsc.subcore_barrier()

        pltpu.sync_copy(deg_sh.at[pl.ds(s * RPT, RPT)], z320_v)
        pltpu.sync_copy(z320_v, deg_out.at[pl.ds(s * RPT, RPT)])

    @pl.when(c == 0)
    def _():
        branch(t2e, dste, g2.at[0], deg_e)

    @pl.when(c == 1)
    def _():
        branch(t2n, dstn, g2.at[1], deg_n)


_sc_phase1 = functools.partial(
    pl.kernel,
    mesh=_MESH,
    out_type=[
        jax.ShapeDtypeStruct((2, NP, D), jnp.float32),
        jax.ShapeDtypeStruct((NP,), jnp.float32),
        jax.ShapeDtypeStruct((NP,), jnp.float32),
    ],
    scratch_types=[
        pltpu.VMEM((GC_PER_TILE, GK), jnp.int32),
        pltpu.VMEM((2, GK, D), jnp.float32),
        pltpu.VMEM((DC_PER_TILE, DK), jnp.int32),
        pltpu.VMEM((DK,), jnp.float32),
        pltpu.VMEM((RPT,), jnp.float32),
        pltpu.VMEM_SHARED((NP,), jnp.float32),
        pltpu.SemaphoreType.DMA,
        pltpu.SemaphoreType.DMA,
    ],
)(_sc_gather_deg)


def _sc_scatter(y2, srce, dste, srcn, dstn,
                acc2,
                six_v, dix_v, rows_v, acc_sh,
                gsem0, gsem1, gsem2, gsem3, ssem0, ssem1, ssem2, ssem3):
    c = lax.axis_index("c")
    s = lax.axis_index("s")
    gsems = [gsem0, gsem1, gsem2, gsem3]
    ssems = [ssem0, ssem1, ssem2, ssem3]

    def fz(i, _):
        for k in range(D // 16):
            rows_v[0, i, pl.ds(k * 16, 16)] = jnp.zeros((16,), jnp.float32)
        return 0
    lax.fori_loop(0, 64, fz, 0)
    for q in range(RPT // 64):
        pltpu.sync_copy(rows_v.at[0, pl.ds(0, 64)],
                        acc_sh.at[pl.ds(s * RPT + q * 64, 64)])

    def branch(y, src_r, dst_r, acc_out):
        pltpu.sync_copy(src_r.at[pl.ds(s * EC_PER_TILE, EC_PER_TILE)], six_v)
        pltpu.sync_copy(dst_r.at[pl.ds(s * EC_PER_TILE, EC_PER_TILE)], dix_v)
        plsc.subcore_barrier()

        pltpu.async_copy(y.at[six_v.at[0]], rows_v.at[0], gsems[0])
        pltpu.async_copy(y.at[six_v.at[1]], rows_v.at[1], gsems[1])
        pltpu.async_copy(y.at[six_v.at[2]], rows_v.at[2], gsems[2])

        def echunk(i, _):
            for b in range(4):
                j = i * 4 + b
                nb = (b + 3) % 4
                pltpu.make_async_copy(y.at[six_v.at[j]], rows_v.at[b],
                                      gsems[b]).wait()
                pltpu.async_copy(rows_v.at[b], acc_sh.at[dix_v.at[j]],
                                 ssems[b], add=True)

                @pl.when(j >= 1)
                def _():
                    pltpu.make_async_copy(rows_v.at[nb],
                                          acc_sh.at[dix_v.at[j - 1]],
                                          ssems[nb]).wait()

                @pl.when(j + 3 < EC_PER_TILE)
                def _():
                    pltpu.async_copy(y.at[six_v.at[j + 3]], rows_v.at[nb],
                                     gsems[nb])
            return 0
        lax.fori_loop(0, EC_PER_TILE // 4, echunk, 0)
        pltpu.make_async_copy(rows_v.at[3],
                              acc_sh.at[dix_v.at[EC_PER_TILE - 1]],
                              ssems[3]).wait()
        plsc.subcore_barrier()

        pltpu.sync_copy(acc_sh.at[pl.ds(s * RPT, RPT)],
                        acc_out.at[pl.ds(s * RPT, RPT)])

    @pl.when(c == 0)
    def _():
        branch(y2.at[0], srce, dste, acc2.at[0])

    @pl.when(c == 1)
    def _():
        branch(y2.at[1], srcn, dstn, acc2.at[1])


_sc_phase2 = functools.partial(
    pl.kernel,
    mesh=_MESH,
    out_type=[
        jax.ShapeDtypeStruct((2, NP, D), jnp.float32),
    ],
    scratch_types=[
        pltpu.VMEM((EC_PER_TILE, EK), jnp.int32),
        pltpu.VMEM((EC_PER_TILE, EK), jnp.int32),
        pltpu.VMEM((4, EK, D), jnp.float32),
        pltpu.VMEM_SHARED((NP, D), jnp.float32),
        pltpu.SemaphoreType.DMA,
        pltpu.SemaphoreType.DMA,
        pltpu.SemaphoreType.DMA,
        pltpu.SemaphoreType.DMA,
        pltpu.SemaphoreType.DMA,
        pltpu.SemaphoreType.DMA,
        pltpu.SemaphoreType.DMA,
        pltpu.SemaphoreType.DMA,
    ],
)(_sc_scatter)


def _mm_body(g_ref, w_ref, deg_ref, y_ref):
    xw = jnp.dot(g_ref[0], w_ref[0], preferred_element_type=jnp.float32)
    dis = lax.rsqrt(deg_ref[0] + 1.0)
    y_ref[0] = xw * dis


def _mm(g2, W2, deg2):
    return pl.pallas_call(
        _mm_body,
        grid=(2, 8),
        in_specs=[
            pl.BlockSpec((1, NP // 8, D), lambda c, i: (c, i, 0)),
            pl.BlockSpec((1, D, D), lambda c, i: (c, 0, 0)),
            pl.BlockSpec((1, NP // 8, 1), lambda c, i: (c, i, 0)),
        ],
        out_specs=pl.BlockSpec((1, NP // 8, D), lambda c, i: (c, i, 0)),
        out_shape=jax.ShapeDtypeStruct((2, NP, D), jnp.float32),
    )(g2, W2, deg2)


_CB = 1000


def _comb_body(te_ref, ye_ref, yn_ref, ae_ref, an_ref, de_ref, dn_ref,
               b2_ref, gg_ref, out_ref):
    i = pl.program_id(0)
    ta = jnp.tanh(gg_ref[0, 0])
    tb = jnp.tanh(gg_ref[0, 1])
    dis_e = lax.rsqrt(de_ref[0] + 1.0)
    dis_n = lax.rsqrt(dn_ref[0] + 1.0)
    ce = dis_e * (ae_ref[0] + ye_ref[0]) + b2_ref[0:1, :]
    cn = dis_n * (an_ref[0] + yn_ref[0]) + b2_ref[1:2, :]
    row = i * _CB + lax.broadcasted_iota(jnp.int32, (_CB, 1), 0)
    out_ref[...] = te_ref[...] + jnp.where(row < N, ta * ce + tb * cn, 0.0)


def _combine(te, y2, acc2, deg2, b2, gg):
    nb = N // _CB
    cap_e = lambda i: (0, jnp.minimum(i, nb - 1), 0)
    cap_n = lambda i: (1, jnp.minimum(i, nb - 1), 0)
    return pl.pallas_call(
        _comb_body,
        grid=(T // _CB,),
        in_specs=[
            pl.BlockSpec((_CB, D), lambda i: (i, 0)),
            pl.BlockSpec((1, _CB, D), cap_e),
            pl.BlockSpec((1, _CB, D), cap_n),
            pl.BlockSpec((1, _CB, D), cap_e),
            pl.BlockSpec((1, _CB, D), cap_n),
            pl.BlockSpec((1, _CB, 1), cap_e),
            pl.BlockSpec((1, _CB, 1), cap_n),
            pl.BlockSpec((2, D), lambda i: (0, 0)),
            pl.BlockSpec((1, 2), lambda i: (0, 0)),
        ],
        out_specs=pl.BlockSpec((_CB, D), lambda i: (i, 0)),
        out_shape=jax.ShapeDtypeStruct((T, D), jnp.float32),
    )(te, y2, y2, acc2, acc2, deg2, deg2, b2, gg)


def kernel(token_embeddings, tokens2edges, edge_index_edges, edges2tokens,
           tokens2nodes, edge_index_nodes, nodes2tokens,
           W_edges, b_edges, W_nodes, b_nodes, gate_a, gate_b):
    te = token_embeddings[0]
    pad = jnp.zeros((NP - N,), jnp.int32)
    t2e = jnp.concatenate([tokens2edges, pad]).reshape(NP // GK, GK)
    t2n = jnp.concatenate([tokens2nodes, pad]).reshape(NP // GK, GK)
    srce = edge_index_edges[0].reshape(EC, EK)
    dste = edge_index_edges[1].reshape(EC, EK)
    srcn = edge_index_nodes[0].reshape(EC, EK)
    dstn = edge_index_nodes[1].reshape(EC, EK)

    dpad = jnp.full((DC * DK - E,), NP - 1, jnp.int32)
    dste_d = jnp.concatenate([edge_index_edges[1], dpad]).reshape(DC, DK)
    dstn_d = jnp.concatenate([edge_index_nodes[1], dpad]).reshape(DC, DK)
    g2, deg_e, deg_n = _sc_phase1(te, t2e, t2n, dste_d, dstn_d)
    deg2 = jnp.stack([deg_e, deg_n])[:, :, None]
    W2 = jnp.stack([W_edges, W_nodes])
    y2 = _mm(g2, W2, deg2)
    (acc2,) = _sc_phase2(y2, srce, dste, srcn, dstn)

    b2 = jnp.stack([b_edges, b_nodes])
    gg = jnp.concatenate([gate_a, gate_b])[None, :]
    out = _combine(te, y2, acc2, deg2, b2, gg)
    return out[None]
